# baseline (speedup 1.0000x reference)
import jax
import jax.numpy as jnp
from jax import lax
from jax.experimental import pallas as pl
from jax.experimental.pallas import tpu as pltpu
from jax.experimental.pallas import tpu_sc as plsc

_B = 4096
_D = 128
_NC = 2
_NS = 16
_NW = _NC * _NS
_BPW = _B // _NW


def _body(ent_hbm, table_hbm, left_hbm, right_hbm,
          stage_sp, lidx_v, ridx_v, lrows_v, rrows_v, gsem, wsem):
    sid = lax.axis_index("s")
    wid = sid * _NC + lax.axis_index("c")
    base = wid * _BPW
    pltpu.sync_copy(ent_hbm.at[pl.ds(base, _BPW)], stage_sp.at[sid])
    pltpu.sync_copy(stage_sp.at[sid, :, 0], lidx_v)
    pltpu.sync_copy(stage_sp.at[sid, :, 1], ridx_v)

    cl = pltpu.async_copy(table_hbm.at[lidx_v], lrows_v, gsem)
    cr = pltpu.async_copy(table_hbm.at[ridx_v], rrows_v, gsem)
    cl.wait()
    wl = pltpu.async_copy(lrows_v, left_hbm.at[pl.ds(base, _BPW)], wsem)
    cr.wait()
    wr = pltpu.async_copy(rrows_v, right_hbm.at[pl.ds(base, _BPW)], wsem)
    wl.wait()
    wr.wait()


_gather = pl.kernel(
    _body,
    out_type=(
        jax.ShapeDtypeStruct((_B, _D), jnp.float32),
        jax.ShapeDtypeStruct((_B, _D), jnp.float32),
    ),
    mesh=plsc.VectorSubcoreMesh(core_axis_name="c", subcore_axis_name="s"),
    scratch_types=[
        pltpu.VMEM_SHARED((_NS, _BPW, 2), jnp.int32),
        pltpu.VMEM((_BPW,), jnp.int32),
        pltpu.VMEM((_BPW,), jnp.int32),
        pltpu.VMEM((_BPW, _D), jnp.float32),
        pltpu.VMEM((_BPW, _D), jnp.float32),
        pltpu.SemaphoreType.DMA,
        pltpu.SemaphoreType.DMA,
    ],
)


def kernel(entity, symbol_emb):
    return _gather(entity.astype(jnp.int32), symbol_emb)


# speedup vs baseline: 1.1040x; 1.1040x over previous
import jax
import jax.numpy as jnp
from jax import lax
from jax.experimental import pallas as pl
from jax.experimental.pallas import tpu as pltpu
from jax.experimental.pallas import tpu_sc as plsc

_B = 4096
_D = 128
_NC = 2
_NS = 16
_NW = _NC * _NS
_ROWS = 2 * _B
_RPW = _ROWS // _NW
_CHUNK = 128
_NCHUNK = _RPW // _CHUNK


def _body(idx_hbm, table_hbm, left_hbm, right_hbm, idx_v, rows_v, gsem, wsem):
    wid = lax.axis_index("s") * _NC + lax.axis_index("c")
    pltpu.sync_copy(idx_hbm.at[wid], idx_v)
    gathers = [
        pltpu.async_copy(
            table_hbm.at[idx_v.at[c]],
            rows_v.at[pl.ds(c * _CHUNK, _CHUNK)],
            gsem,
        )
        for c in range(_NCHUNK)
    ]

    half = _NW // 2

    def drain(out_hbm, base):
        writes = []
        for c in range(_NCHUNK):
            gathers[c].wait()
            writes.append(pltpu.async_copy(
                rows_v.at[pl.ds(c * _CHUNK, _CHUNK)],
                out_hbm.at[pl.ds(base + c * _CHUNK, _CHUNK)],
                wsem,
            ))
        for w in writes:
            w.wait()

    @pl.when(wid < half)
    def _():
        drain(left_hbm, wid * _RPW)

    @pl.when(wid >= half)
    def _():
        drain(right_hbm, (wid - half) * _RPW)


_gather = pl.kernel(
    _body,
    out_type=(
        jax.ShapeDtypeStruct((_B, _D), jnp.float32),
        jax.ShapeDtypeStruct((_B, _D), jnp.float32),
    ),
    mesh=plsc.VectorSubcoreMesh(core_axis_name="c", subcore_axis_name="s"),
    scratch_types=[
        pltpu.VMEM((_NCHUNK, _CHUNK), jnp.int32),
        pltpu.VMEM((_RPW, _D), jnp.float32),
        pltpu.SemaphoreType.DMA,
        pltpu.SemaphoreType.DMA,
    ],
    compiler_params=pltpu.CompilerParams(skip_device_barrier=True),
)


def kernel(entity, symbol_emb):
    idx = entity.astype(jnp.int32).T.reshape(_NW, _NCHUNK, _CHUNK)
    return _gather(idx, symbol_emb)


# speedup vs baseline: 1.1076x; 1.0032x over previous
import jax
import jax.numpy as jnp
from jax import lax
from jax.experimental import pallas as pl
from jax.experimental.pallas import tpu as pltpu
from jax.experimental.pallas import tpu_sc as plsc

_B = 4096
_D = 128
_NC = 2
_NS = 16
_NW = _NC * _NS
_ROWS = 2 * _B
_RPW = _ROWS // _NW
_CHUNK = 128
_NCHUNK = _RPW // _CHUNK


def _body(idx_hbm, table_hbm, left_hbm, right_hbm, idx_v, rows_v, sem):
    wid = lax.axis_index("s") * _NC + lax.axis_index("c")
    pltpu.sync_copy(idx_hbm.at[wid], idx_v)
    gathers = [
        pltpu.async_copy(
            table_hbm.at[idx_v.at[c]],
            rows_v.at[pl.ds(c * _CHUNK, _CHUNK)],
            sem,
        )
        for c in range(_NCHUNK)
    ]
    for g in gathers:
        g.wait()

    half = _NW // 2

    @pl.when(wid < half)
    def _():
        pltpu.sync_copy(rows_v, left_hbm.at[pl.ds(wid * _RPW, _RPW)])

    @pl.when(wid >= half)
    def _():
        pltpu.sync_copy(rows_v, right_hbm.at[pl.ds((wid - half) * _RPW, _RPW)])


_gather = pl.kernel(
    _body,
    out_type=(
        jax.ShapeDtypeStruct((_B, _D), jnp.float32),
        jax.ShapeDtypeStruct((_B, _D), jnp.float32),
    ),
    mesh=plsc.VectorSubcoreMesh(core_axis_name="c", subcore_axis_name="s"),
    scratch_types=[
        pltpu.VMEM((_NCHUNK, _CHUNK), jnp.int32),
        pltpu.VMEM((_RPW, _D), jnp.float32),
        pltpu.SemaphoreType.DMA,
    ],
)


def kernel(entity, symbol_emb):
    idx = entity.astype(jnp.int32).T.reshape(_NW, _NCHUNK, _CHUNK)
    idx = lax.optimization_barrier(idx)
    return _gather(idx, symbol_emb)


# speedup vs baseline: 1.1117x; 1.0037x over previous
import jax
import jax.numpy as jnp
from jax import lax
from jax.experimental import pallas as pl
from jax.experimental.pallas import tpu as pltpu
from jax.experimental.pallas import tpu_sc as plsc

_B = 4096
_D = 128
_NC = 2
_NS = 16
_NW = _NC * _NS
_ROWS = 2 * _B
_RPW = _ROWS // _NW
_CHUNK = 128
_NCHUNK = _RPW // _CHUNK


def _body(idx_hbm, table_hbm, left_hbm, right_hbm, idx_v, rows_v, sem):
    wid = lax.axis_index("s") * _NC + lax.axis_index("c")
    pltpu.sync_copy(idx_hbm.at[wid], idx_v)
    gathers = [
        pltpu.async_copy(
            table_hbm.at[idx_v.at[c]],
            rows_v.at[pl.ds(c * _CHUNK, _CHUNK)],
            sem,
        )
        for c in range(_NCHUNK)
    ]
    for g in gathers:
        g.wait()

    half = _NW // 2

    @pl.when(wid < half)
    def _():
        pltpu.sync_copy(rows_v, left_hbm.at[pl.ds(wid * _RPW, _RPW)])

    @pl.when(wid >= half)
    def _():
        pltpu.sync_copy(rows_v, right_hbm.at[pl.ds((wid - half) * _RPW, _RPW)])


_gather = pl.kernel(
    _body,
    out_type=(
        jax.ShapeDtypeStruct((_B, _D), jnp.float32),
        jax.ShapeDtypeStruct((_B, _D), jnp.float32),
    ),
    mesh=plsc.VectorSubcoreMesh(core_axis_name="c", subcore_axis_name="s"),
    scratch_types=[
        pltpu.VMEM((_NCHUNK, _CHUNK), jnp.int32),
        pltpu.VMEM((_RPW, _D), jnp.float32),
        pltpu.SemaphoreType.DMA,
    ],
    compiler_params=pltpu.CompilerParams(allow_input_fusion=[True, False]),
)


def kernel(entity, symbol_emb):
    idx = entity.astype(jnp.int32).T.reshape(_NW, _NCHUNK, _CHUNK)
    idx = lax.optimization_barrier(idx)
    return _gather(idx, symbol_emb)
